# baseline (speedup 1.0000x reference)
import functools

import jax
import jax.numpy as jnp
from jax import lax
from jax.experimental import pallas as pl
from jax.experimental.pallas import tpu as pltpu
from jax.experimental.pallas import tpu_sc as plsc

NC = 2
NS = 16
NW = NC * NS
LANE = 128
CK = 64


def _chunk_sizes(tpw):
    sizes = [CK] * (tpw // CK)
    if tpw % CK:
        sizes.append(tpw % CK)
    return sizes


def _sc_gather_build(B, P, D, n_tok, nrows, with_prompt):
    tpw = n_tok // NW
    sizes = _chunk_sizes(tpw)
    p8 = 8 * ((P + 7) // 8)
    mesh = plsc.VectorSubcoreMesh(core_axis_name="c", subcore_axis_name="s")

    scratch = []
    for sz in sorted(set(sizes)):
        scratch += [
            pltpu.VMEM((sz,), jnp.int32),
            pltpu.VMEM((sz,), jnp.int32),
            pltpu.VMEM((sz, D), jnp.float32),
        ]
    if with_prompt:
        scratch += [
            pltpu.VMEM((p8,), jnp.int32),
            pltpu.VMEM((p8, D), jnp.float32),
        ]
    scratch.append(pltpu.SemaphoreType.DMA)

    size_order = sorted(set(sizes))

    @functools.partial(
        pl.kernel,
        mesh=mesh,
        out_type=jax.ShapeDtypeStruct((nrows, D), jnp.float32),
        scratch_types=scratch,
        compiler_params=pltpu.CompilerParams(use_tc_tiling_on_sc=True),
    )
    def sc_gather(ids_hbm, table_hbm, sp_hbm, dpos_hbm, pidx_hbm, x_hbm,
                  *refs):
        sem = refs[-1]
        bufs = {}
        for i, sz in enumerate(size_order):
            bufs[sz] = (refs[3 * i], refs[3 * i + 1], refs[3 * i + 2])
        wid = lax.axis_index("s") * NC + lax.axis_index("c")

        if with_prompt:
            pidx_v, sp_v = refs[-3], refs[-2]
            wpb = NW // B

            @pl.when(wid % wpb == 0)
            def _():
                batch = wid // wpb
                pltpu.sync_copy(sp_hbm, sp_v)
                pltpu.sync_copy(pidx_hbm.at[batch], pidx_v)
                pltpu.async_copy(sp_v, x_hbm.at[pidx_v], sem).wait()

        src_base = wid * tpw
        off = 0
        for sz in sizes:
            idx_v, didx_v, rows_v = bufs[sz]
            pltpu.sync_copy(ids_hbm.at[pl.ds(src_base + off, sz)], idx_v)
            pltpu.sync_copy(dpos_hbm.at[pl.ds(src_base + off, sz)], didx_v)
            pltpu.async_copy(table_hbm.at[idx_v], rows_v, sem).wait()
            pltpu.async_copy(rows_v, x_hbm.at[didx_v], sem).wait()
            off += sz

    return sc_gather


def _tc_matmul_build(B, R, D, tile, jo, nj, alias):
    nt = D // LANE
    rt = tile // B

    def body(*refs):
        if alias:
            x_ref, w_ref, b_ref, m_ref, _, o_ref = refs
        else:
            x_ref, w_ref, b_ref, m_ref, o_ref = refs
        acc = jnp.dot(x_ref[...], w_ref[...],
                      preferred_element_type=jnp.float32)
        h = jnp.tanh(acc + b_ref[...]) * m_ref[...]
        o_ref[...] = h.reshape(rt, B, nt, LANE).transpose(0, 2, 1, 3).reshape(
            rt, nt * B, LANE)

    in_specs = [
        pl.BlockSpec((tile, D), lambda j: (j, 0)),
        pl.BlockSpec((D, D), lambda j: (0, 0)),
        pl.BlockSpec((1, D), lambda j: (0, 0)),
        pl.BlockSpec((tile, 1), lambda j: (jo + j, 0)),
    ]
    kwargs = {}
    if alias:
        in_specs.append(pl.BlockSpec(memory_space=pl.ANY))
        kwargs["input_output_aliases"] = {4: 0}

    return pl.pallas_call(
        body,
        grid=(nj,),
        in_specs=in_specs,
        out_specs=pl.BlockSpec((rt, nt * B, LANE), lambda j: (jo + j, 0, 0)),
        out_shape=jax.ShapeDtypeStruct((R, nt * B, LANE), jnp.float32),
        compiler_params=pltpu.CompilerParams(
            dimension_semantics=("arbitrary",),
        ),
        **kwargs,
    )


def kernel(input_ids, attention_mask, token_type_ids, word_embeddings,
           soft_prompt, W, b):
    B, S = input_ids.shape
    V, D = word_embeddings.shape
    P = soft_prompt.shape[0]
    R = P + S
    p8 = 8 * ((P + 7) // 8)
    tile = 512
    rt = tile // B
    nj0 = (R // rt + 1) // 2 + 1
    pos_mid = nj0 * rt
    nj1 = (R + rt - 1) // rt - nj0
    s_mid = pos_mid - P

    def pad32(n):
        q = NW * 8
        return q * ((n + q - 1) // q)

    n0, n1 = B * s_mid, B * (S - s_mid)
    n0p, n1p = pad32(n0), pad32(n1)
    rows0 = pos_mid * B
    rows1 = (nj0 + nj1) * rt * B - rows0
    dump0, dump1 = rows0, rows1

    ids0 = input_ids[:, :s_mid].reshape(-1).astype(jnp.int32)
    ids1 = input_ids[:, s_mid:].reshape(-1).astype(jnp.int32)
    ids0 = jnp.pad(ids0, (0, n0p - n0))
    ids1 = jnp.pad(ids1, (0, n1p - n1))

    bcol = jnp.arange(B, dtype=jnp.int32)[:, None]
    s0 = jnp.arange(s_mid, dtype=jnp.int32)[None, :]
    s1 = jnp.arange(S - s_mid, dtype=jnp.int32)[None, :]
    dpos0 = ((P + s0) * B + bcol).reshape(-1)
    dpos1 = (s1 * B + bcol).reshape(-1)
    dpos0 = jnp.pad(dpos0, (0, n0p - n0), constant_values=dump0)
    dpos1 = jnp.pad(dpos1, (0, n1p - n1), constant_values=dump1)

    sp_pad = jnp.pad(soft_prompt, ((0, p8 - P), (0, 0)))
    prow = jnp.arange(p8, dtype=jnp.int32)[None, :]
    pidx = jnp.where(prow < P, prow * B + bcol, dump0)

    sc0 = _sc_gather_build(B, P, D, n0p, rows0 + 8, True)
    sc1 = _sc_gather_build(B, P, D, n1p, rows1 + 8, False)
    x0 = sc0(ids0, word_embeddings, sp_pad, dpos0, pidx)
    x1 = sc1(ids1, word_embeddings, sp_pad, dpos1, pidx)

    mask = jnp.concatenate(
        [jnp.ones((B, P), dtype=attention_mask.dtype), attention_mask], axis=1
    ).astype(jnp.float32).T.reshape(R * B, 1)

    tc0 = _tc_matmul_build(B, R, D, tile, 0, nj0, False)
    tc1 = _tc_matmul_build(B, R, D, tile, nj0, nj1, True)
    out_a = tc0(x0, W, b.reshape(1, D), mask)
    out3 = tc1(x1, W, b.reshape(1, D), mask, out_a)
    return (out3.reshape(R, D // LANE, B, LANE)
            .transpose(2, 0, 1, 3).reshape(B, R, D))


# speedup vs baseline: 1.1338x; 1.1338x over previous
import functools

import jax
import jax.numpy as jnp
from jax import lax
from jax.experimental import pallas as pl
from jax.experimental.pallas import tpu as pltpu
from jax.experimental.pallas import tpu_sc as plsc

NC = 2
NS = 16
NW = NC * NS


def _sc_gather_build(B, S, P, V, D):
    tokens = B * S
    tpw = tokens // NW
    ck = 56
    sizes = [ck] * (tpw // ck)
    if tpw % ck:
        sizes.append(tpw % ck)
    tail = sizes[-1] if sizes[-1] != ck else None
    wpb = NW // B
    p8 = 8 * ((P + 7) // 8)
    mesh = plsc.VectorSubcoreMesh(core_axis_name="c", subcore_axis_name="s")

    @functools.partial(
        pl.kernel,
        mesh=mesh,
        out_type=jax.ShapeDtypeStruct(((P + S) * B + 8, D), jnp.float32),
        scratch_types=[
            pltpu.VMEM((ck,), jnp.int32),
            pltpu.VMEM((ck,), jnp.int32),
            pltpu.VMEM((ck,), jnp.int32),
            pltpu.VMEM((ck,), jnp.int32),
            pltpu.VMEM((ck, D), jnp.float32),
            pltpu.VMEM((ck, D), jnp.float32),
            pltpu.VMEM((p8,), jnp.int32),
            pltpu.SemaphoreType.DMA,
            pltpu.SemaphoreType.DMA,
            pltpu.SemaphoreType.DMA,
            pltpu.SemaphoreType.DMA,
        ],
        compiler_params=pltpu.CompilerParams(use_tc_tiling_on_sc=True),
    )
    def sc_gather(ids_hbm, table_hbm, sp_hbm, dpos_hbm, pidx_hbm, x_hbm,
                  idx_a, idx_b, didx_a, didx_b, rows_a, rows_b, pidx_v,
                  sem_ga, sem_gb, sem_sa, sem_sb):
        wid = lax.axis_index("s") * NC + lax.axis_index("c")
        idx = [idx_a, idx_b]
        didx = [didx_a, didx_b]
        rows = [rows_a, rows_b]
        sem_g = [sem_ga, sem_gb]
        sem_s = [sem_sa, sem_sb]

        @pl.when(wid % wpb == 0)
        def _():
            batch = wid // wpb
            pltpu.sync_copy(sp_hbm, rows_a.at[pl.ds(0, p8)])
            pltpu.sync_copy(pidx_hbm.at[batch], pidx_v)
            pltpu.async_copy(rows_a.at[pl.ds(0, p8)], x_hbm.at[pidx_v],
                             sem_sa).wait()

        src_base = wid * tpw

        def bufs(k, sz):
            par = k % 2
            if sz == ck:
                return (idx[par], didx[par], rows[par], sem_g[par],
                        sem_s[par])
            return (idx[par].at[pl.ds(0, sz)], didx[par].at[pl.ds(0, sz)],
                    rows[par].at[pl.ds(0, sz)], sem_g[par], sem_s[par])

        offs = []
        o = 0
        for sz in sizes:
            offs.append(o)
            o += sz

        i0, d0, _, _, _ = bufs(0, sizes[0])
        pltpu.sync_copy(ids_hbm.at[pl.ds(src_base, sizes[0])], i0)
        pltpu.sync_copy(dpos_hbm.at[pl.ds(src_base, sizes[0])], d0)

        h_s = [None, None]
        for k, (off, sz) in enumerate(zip(offs, sizes)):
            par = k % 2
            i_r, d_r, r_r, sg, ss = bufs(k, sz)
            h_g = pltpu.async_copy(table_hbm.at[i_r], r_r, sg)
            if k + 1 < len(sizes):
                npar = (k + 1) % 2
                if h_s[npar] is not None:
                    h_s[npar].wait()
                    h_s[npar] = None
                ni, nd, _, _, _ = bufs(k + 1, sizes[k + 1])
                nxt = offs[k + 1]
                pltpu.sync_copy(
                    ids_hbm.at[pl.ds(src_base + nxt, sizes[k + 1])], ni)
                pltpu.sync_copy(
                    dpos_hbm.at[pl.ds(src_base + nxt, sizes[k + 1])], nd)
            h_g.wait()
            h_s[par] = pltpu.async_copy(r_r, x_hbm.at[d_r], ss)
        for h in h_s:
            if h is not None:
                h.wait()

    return sc_gather


def _tc_matmul_build(B, R, D, tile):
    lane = 128
    nt = D // lane
    rt = tile // B
    nj = (R + rt - 1) // rt

    def body(x_ref, w_ref, b_ref, m_ref, o_ref):
        acc = jnp.dot(x_ref[...].astype(jnp.bfloat16),
                      w_ref[...].astype(jnp.bfloat16),
                      preferred_element_type=jnp.float32)
        h = jnp.tanh(acc + b_ref[...]) * m_ref[...]
        o_ref[...] = h.reshape(rt, B, nt, lane).transpose(0, 2, 1, 3).reshape(
            rt, nt * B, lane)

    return pl.pallas_call(
        body,
        grid=(nj,),
        in_specs=[
            pl.BlockSpec((tile, D), lambda j: (j, 0)),
            pl.BlockSpec((D, D), lambda j: (0, 0)),
            pl.BlockSpec((1, D), lambda j: (0, 0)),
            pl.BlockSpec((tile, 1), lambda j: (j, 0)),
        ],
        out_specs=pl.BlockSpec((rt, nt * B, lane), lambda j: (j, 0, 0)),
        out_shape=jax.ShapeDtypeStruct((R, nt * B, lane), jnp.float32),
        compiler_params=pltpu.CompilerParams(
            dimension_semantics=("arbitrary",),
        ),
    )


def kernel(input_ids, attention_mask, token_type_ids, word_embeddings,
           soft_prompt, W, b):
    B, S = input_ids.shape
    V, D = word_embeddings.shape
    P = soft_prompt.shape[0]
    p8 = 8 * ((P + 7) // 8)

    ids = input_ids.reshape(-1).astype(jnp.int32)
    sp_pad = jnp.pad(soft_prompt, ((0, p8 - P), (0, 0)))
    dump = (P + S) * B
    dpos = ((P + jnp.arange(S, dtype=jnp.int32))[None, :] * B
            + jnp.arange(B, dtype=jnp.int32)[:, None]).reshape(-1)
    prow = jnp.arange(p8, dtype=jnp.int32)[None, :]
    pidx = jnp.where(prow < P,
                     prow * B + jnp.arange(B, dtype=jnp.int32)[:, None],
                     dump)

    sc_gather = _sc_gather_build(B, S, P, V, D)
    x = sc_gather(ids, word_embeddings, sp_pad, dpos, pidx)

    mask = jnp.concatenate(
        [jnp.ones((B, P), dtype=attention_mask.dtype), attention_mask], axis=1
    ).astype(jnp.float32).T.reshape((P + S) * B, 1)

    tc = _tc_matmul_build(B, P + S, D, 512)
    out3 = tc(x, W, b.reshape(1, D), mask)
    lane = 128
    return (out3.reshape(P + S, D // lane, B, lane)
            .transpose(2, 0, 1, 3).reshape(B, P + S, D))
